# baseline (speedup 1.0000x reference)
import jax
import jax.numpy as jnp
from jax import lax
from jax.experimental import pallas as pl
from jax.experimental.pallas import tpu as pltpu

_B = 256
_D = 4 * 64 * 64
_T = 1000
_BF = 8192


def _tc_fused(sac2, s1mac2, t2, xt, nt):
    grid = (_D // _BF,)

    def body(sac_ref, s1mac_ref, t_ref, x_ref, n_ref, o_ref, c1_ref, c2_ref):
        @pl.when(pl.program_id(0) == 0)
        def _():
            tt = t_ref[...]
            rows = lax.broadcasted_iota(jnp.int32, (_T, _B), 0)
            onehot = jnp.where(rows == tt, 1.0, 0.0)
            dn = (((1,), (0,)), ((), ()))
            c1_ref[...] = lax.dot_general(
                sac_ref[...], onehot, dimension_numbers=dn,
                preferred_element_type=jnp.float32,
                precision=lax.Precision.HIGHEST)
            c2_ref[...] = lax.dot_general(
                s1mac_ref[...], onehot, dimension_numbers=dn,
                preferred_element_type=jnp.float32,
                precision=lax.Precision.HIGHEST)

        o_ref[...] = c1_ref[...] * x_ref[...] + c2_ref[...] * n_ref[...]

    return pl.pallas_call(
        body,
        grid=grid,
        in_specs=[
            pl.BlockSpec((1, _T), lambda i: (0, 0)),
            pl.BlockSpec((1, _T), lambda i: (0, 0)),
            pl.BlockSpec((1, _B), lambda i: (0, 0)),
            pl.BlockSpec((_BF, _B), lambda i: (i, 0)),
            pl.BlockSpec((_BF, _B), lambda i: (i, 0)),
        ],
        out_specs=pl.BlockSpec((_BF, _B), lambda i: (i, 0)),
        out_shape=jax.ShapeDtypeStruct((_D, _B), jnp.float32),
        scratch_shapes=[
            pltpu.VMEM((1, _B), jnp.float32),
            pltpu.VMEM((1, _B), jnp.float32),
        ],
        compiler_params=pltpu.CompilerParams(
            dimension_semantics=("arbitrary",)),
    )(sac2, s1mac2, t2, xt, nt)


_CHUNKS = (1024, 2048, 4096, 4096, 4096, 1024)
_OFFS = tuple(sum(_CHUNKS[:i]) for i in range(len(_CHUNKS)))
_SLOTS = 3
_MAXCH = max(_CHUNKS)


def _tc_fused_ramp(sac2, s1mac2, t2, xt, nt):
    def body(sac_ref, s1mac_ref, t_ref, x_hbm, n_hbm, o_hbm,
             xb, nb, ob, sx, sn, so, c1_ref, c2_ref):
        def fetch(s):
            sl = s % _SLOTS
            ch, off = _CHUNKS[s], _OFFS[s]
            pltpu.make_async_copy(
                x_hbm.at[pl.ds(off, ch), :],
                xb.at[sl, pl.ds(0, ch), :], sx.at[sl]).start()
            pltpu.make_async_copy(
                n_hbm.at[pl.ds(off, ch), :],
                nb.at[sl, pl.ds(0, ch), :], sn.at[sl]).start()

        def wait_fetch(s):
            sl = s % _SLOTS
            ch, off = _CHUNKS[s], _OFFS[s]
            pltpu.make_async_copy(
                x_hbm.at[pl.ds(off, ch), :],
                xb.at[sl, pl.ds(0, ch), :], sx.at[sl]).wait()
            pltpu.make_async_copy(
                n_hbm.at[pl.ds(off, ch), :],
                nb.at[sl, pl.ds(0, ch), :], sn.at[sl]).wait()

        def store(s):
            sl = s % _SLOTS
            ch, off = _CHUNKS[s], _OFFS[s]
            pltpu.make_async_copy(
                ob.at[sl, pl.ds(0, ch), :],
                o_hbm.at[pl.ds(off, ch), :], so.at[sl]).start()

        def wait_store(s):
            sl = s % _SLOTS
            ch, off = _CHUNKS[s], _OFFS[s]
            pltpu.make_async_copy(
                ob.at[sl, pl.ds(0, ch), :],
                o_hbm.at[pl.ds(off, ch), :], so.at[sl]).wait()

        fetch(0)
        fetch(1)
        tt = t_ref[...]
        rows = lax.broadcasted_iota(jnp.int32, (_T, _B), 0)
        onehot = jnp.where(rows == tt, 1.0, 0.0)
        dn = (((1,), (0,)), ((), ()))
        c1_ref[...] = lax.dot_general(
            sac_ref[...], onehot, dimension_numbers=dn,
            preferred_element_type=jnp.float32,
            precision=lax.Precision.HIGHEST)
        c2_ref[...] = lax.dot_general(
            s1mac_ref[...], onehot, dimension_numbers=dn,
            preferred_element_type=jnp.float32,
            precision=lax.Precision.HIGHEST)
        c1 = c1_ref[...]
        c2 = c2_ref[...]
        ns = len(_CHUNKS)
        for s in range(ns):
            sl = s % _SLOTS
            ch = _CHUNKS[s]
            wait_fetch(s)
            if s + 2 < ns:
                fetch(s + 2)
            if s >= _SLOTS:
                wait_store(s - _SLOTS)
            ob[sl, pl.ds(0, ch), :] = c1 * xb[sl, pl.ds(0, ch), :] \
                + c2 * nb[sl, pl.ds(0, ch), :]
            store(s)
        for s in range(max(0, ns - _SLOTS), ns):
            wait_store(s)

    return pl.pallas_call(
        body,
        in_specs=[
            pl.BlockSpec(memory_space=pltpu.MemorySpace.VMEM),
            pl.BlockSpec(memory_space=pltpu.MemorySpace.VMEM),
            pl.BlockSpec(memory_space=pltpu.MemorySpace.VMEM),
            pl.BlockSpec(memory_space=pl.ANY),
            pl.BlockSpec(memory_space=pl.ANY),
        ],
        out_specs=pl.BlockSpec(memory_space=pl.ANY),
        out_shape=jax.ShapeDtypeStruct((_D, _B), jnp.float32),
        scratch_shapes=[
            pltpu.VMEM((_SLOTS, _MAXCH, _B), jnp.float32),
            pltpu.VMEM((_SLOTS, _MAXCH, _B), jnp.float32),
            pltpu.VMEM((_SLOTS, _MAXCH, _B), jnp.float32),
            pltpu.SemaphoreType.DMA((_SLOTS,)),
            pltpu.SemaphoreType.DMA((_SLOTS,)),
            pltpu.SemaphoreType.DMA((_SLOTS,)),
            pltpu.VMEM((1, _B), jnp.float32),
            pltpu.VMEM((1, _B), jnp.float32),
        ],
    )(sac2, s1mac2, t2, xt, nt)


def kernel(x_start, t, noise, sqrt_alphas_cumprod, sqrt_one_minus_alphas_cumprod):
    B, C, H, W = x_start.shape
    xt = jnp.transpose(x_start, (1, 2, 3, 0)).reshape(C * H * W, B)
    nt = jnp.transpose(noise, (1, 2, 3, 0)).reshape(C * H * W, B)
    ot = _tc_fused_ramp(sqrt_alphas_cumprod.reshape(1, _T),
                   sqrt_one_minus_alphas_cumprod.reshape(1, _T),
                   t.reshape(1, B), xt, nt)
    return jnp.transpose(ot.reshape(C, H, W, B), (3, 0, 1, 2))


# speedup vs baseline: 1.0896x; 1.0896x over previous
import jax
import jax.numpy as jnp
from jax import lax
from jax.experimental import pallas as pl
from jax.experimental.pallas import tpu as pltpu

_B = 256
_D = 4 * 64 * 64
_T = 1000
_BF = 8192


def _tc_fused(sac2, s1mac2, t2, xt, nt):
    grid = (_D // _BF,)

    def body(sac_ref, s1mac_ref, t_ref, x_ref, n_ref, o_ref, c1_ref, c2_ref):
        @pl.when(pl.program_id(0) == 0)
        def _():
            tt = t_ref[...]
            rows = lax.broadcasted_iota(jnp.int32, (_T, _B), 0)
            onehot = jnp.where(rows == tt, 1.0, 0.0)
            dn = (((1,), (0,)), ((), ()))
            c1_ref[...] = lax.dot_general(
                sac_ref[...], onehot, dimension_numbers=dn,
                preferred_element_type=jnp.float32,
                precision=lax.Precision.HIGHEST)
            c2_ref[...] = lax.dot_general(
                s1mac_ref[...], onehot, dimension_numbers=dn,
                preferred_element_type=jnp.float32,
                precision=lax.Precision.HIGHEST)

        o_ref[...] = c1_ref[...] * x_ref[...] + c2_ref[...] * n_ref[...]

    return pl.pallas_call(
        body,
        grid=grid,
        in_specs=[
            pl.BlockSpec((1, _T), lambda i: (0, 0)),
            pl.BlockSpec((1, _T), lambda i: (0, 0)),
            pl.BlockSpec((1, _B), lambda i: (0, 0)),
            pl.BlockSpec((_BF, _B), lambda i: (i, 0)),
            pl.BlockSpec((_BF, _B), lambda i: (i, 0)),
        ],
        out_specs=pl.BlockSpec((_BF, _B), lambda i: (i, 0)),
        out_shape=jax.ShapeDtypeStruct((_D, _B), jnp.float32),
        scratch_shapes=[
            pltpu.VMEM((1, _B), jnp.float32),
            pltpu.VMEM((1, _B), jnp.float32),
        ],
        compiler_params=pltpu.CompilerParams(
            dimension_semantics=("arbitrary",)),
    )(sac2, s1mac2, t2, xt, nt)


def kernel(x_start, t, noise, sqrt_alphas_cumprod, sqrt_one_minus_alphas_cumprod):
    B, C, H, W = x_start.shape
    xt = jnp.transpose(x_start, (1, 2, 3, 0)).reshape(C * H * W, B)
    nt = jnp.transpose(noise, (1, 2, 3, 0)).reshape(C * H * W, B)
    ot = _tc_fused(sqrt_alphas_cumprod.reshape(1, _T),
                   sqrt_one_minus_alphas_cumprod.reshape(1, _T),
                   t.reshape(1, B), xt, nt)
    return jnp.transpose(ot.reshape(C, H, W, B), (3, 0, 1, 2))
